# Initial kernel scaffold; baseline (speedup 1.0000x reference)
import jax
import jax.numpy as jnp
from jax.experimental import pallas as pl


def kernel(x, edge_index, edge_type, weight, comp, bias):
    raise NotImplementedError("write your pallas kernel here")


# speedup vs baseline: 15.1416x; 15.1416x over previous
import jax
import jax.numpy as jnp
from jax import lax
from jax.experimental import pallas as pl
from jax.experimental.pallas import tpu as pltpu
from jax.experimental.pallas import tpu_sc as plsc

N = 10000
E = 320000
D = 128
R = 16
B = 4

NC = 2
NS = 16
H = D // NC
EPT = E // NS
CH = 128
NCHUNK = EPT // CH
REM = EPT - NCHUNK * CH
NP = 10240
RPT = NP // NS


def _mm_body(x_ref, w_ref, comp_ref, z_ref, zroot_ref):
    xb = x_ref[...]
    ys = [jnp.dot(xb, w_ref[b], preferred_element_type=jnp.float32,
                  precision=lax.Precision.HIGHEST)
          for b in range(B)]
    zroot_ref[...] = jnp.dot(xb, w_ref[B], preferred_element_type=jnp.float32,
                             precision=lax.Precision.HIGHEST)
    accs = []
    for r in range(R):
        acc = comp_ref[r, 0] * ys[0]
        for b in range(1, B):
            acc = acc + comp_ref[r, b] * ys[b]
        accs.append(acc)
    for h in range(NC):
        for r in range(0, R, 2):
            blkv = jnp.concatenate(
                [accs[r][:, h * H:(h + 1) * H],
                 accs[r + 1][:, h * H:(h + 1) * H]], axis=1)
            z_ref[h, :, r * H:(r + 2) * H] = blkv


def _relation_transform(x, weight, comp):
    blk = 1000
    grid = (N // blk,)
    return pl.pallas_call(
        _mm_body,
        grid=grid,
        in_specs=[
            pl.BlockSpec((blk, D), lambda i: (i, 0)),
            pl.BlockSpec((B + 1, D, D), lambda i: (0, 0, 0)),
            pl.BlockSpec(memory_space=pltpu.SMEM),
        ],
        out_specs=[
            pl.BlockSpec((NC, blk, R * H), lambda i: (0, i, 0)),
            pl.BlockSpec((blk, D), lambda i: (i, 0)),
        ],
        out_shape=[
            jax.ShapeDtypeStruct((NC, N, R * H), jnp.float32),
            jax.ShapeDtypeStruct((N, D), jnp.float32),
        ],
    )(x, weight, comp)


def _sc_body(zmsg, src3, typ3, dst3, srcr, typr, dstr, zrows, zdeg, ones,
             out_p, out_d,
             srcb, typb, dstb, r0, r1, rrem, sb32, tb32, db32, onesb,
             acc, dacc, semg0, semg1):
    c = lax.axis_index("c")
    s = lax.axis_index("s")
    cbase = c * (N * R)

    pltpu.sync_copy(zrows, acc.at[pl.ds(s * RPT, RPT)])
    pltpu.sync_copy(zdeg, dacc.at[pl.ds(s * RPT, RPT)])

    pltpu.sync_copy(ones, onesb)
    pltpu.sync_copy(src3.at[s], srcb)
    pltpu.sync_copy(typ3.at[s], typb)
    pltpu.sync_copy(dst3.at[s], dstb)
    plsc.subcore_barrier()

    def cidx(j, carry):
        for k in range(8):
            sl = pl.ds(k * 16, 16)
            srcb[j, sl] = srcb[j, sl] * R + typb[j, sl] + cbase
        return carry
    lax.fori_loop(0, NCHUNK, cidx, 0)

    pltpu.async_copy(zmsg.at[srcb.at[0]], r0, semg0)

    def step(i, carry):
        j0 = 2 * i
        j1 = 2 * i + 1
        pltpu.make_async_copy(zmsg.at[srcb.at[j0]], r0, semg0).wait()
        pltpu.async_copy(zmsg.at[srcb.at[j1]], r1, semg1)
        pltpu.sync_copy(r0, acc.at[dstb.at[j0]], add=True)

        @pl.when(c == 0)
        def _():
            pltpu.sync_copy(onesb, dacc.at[dstb.at[j0]], add=True)

        pltpu.make_async_copy(zmsg.at[srcb.at[j1]], r1, semg1).wait()

        @pl.when(j1 + 1 < NCHUNK)
        def _():
            pltpu.async_copy(zmsg.at[srcb.at[j1 + 1]], r0, semg0)

        pltpu.sync_copy(r1, acc.at[dstb.at[j1]], add=True)

        @pl.when(c == 0)
        def _():
            pltpu.sync_copy(onesb, dacc.at[dstb.at[j1]], add=True)
        return carry
    lax.fori_loop(0, NCHUNK // 2, step, 0)

    pltpu.sync_copy(srcr.at[s], sb32)
    pltpu.sync_copy(typr.at[s], tb32)
    pltpu.sync_copy(dstr.at[s], db32)
    for q in range(REM // 16):
        sl = pl.ds(q * 16, 16)
        idxv = sb32[sl] * R + tb32[sl] + cbase
        pltpu.async_copy(zmsg.at[idxv], rrem, semg0).wait()
        dv = db32[sl]
        pltpu.sync_copy(rrem, acc.at[dv], add=True)

        @pl.when(c == 0)
        def _():
            pltpu.sync_copy(onesb.at[pl.ds(0, 16)], dacc.at[dv], add=True)

    plsc.subcore_barrier()
    pltpu.sync_copy(acc.at[pl.ds(s * RPT, RPT)],
                    out_p.at[c].at[pl.ds(s * RPT, RPT)])

    @pl.when(c == 0)
    def _():
        pltpu.sync_copy(dacc.at[pl.ds(s * RPT, RPT)],
                        out_d.at[pl.ds(s * RPT, RPT)])


def _sc_aggregate(zmsg, src3, typ3, dst3, srcr, typr, dstr, zrows, zdeg, ones):
    mesh = plsc.VectorSubcoreMesh(core_axis_name="c", subcore_axis_name="s")
    kern = pl.kernel(
        _sc_body,
        out_type=(
            jax.ShapeDtypeStruct((NC, NP, H), jnp.float32),
            jax.ShapeDtypeStruct((NP, 16), jnp.float32),
        ),
        mesh=mesh,
        scratch_types=(
            pltpu.VMEM((NCHUNK, CH), jnp.int32),
            pltpu.VMEM((NCHUNK, CH), jnp.int32),
            pltpu.VMEM((NCHUNK, CH), jnp.int32),
            pltpu.VMEM((CH, H), jnp.float32),
            pltpu.VMEM((CH, H), jnp.float32),
            pltpu.VMEM((16, H), jnp.float32),
            pltpu.VMEM((REM,), jnp.int32),
            pltpu.VMEM((REM,), jnp.int32),
            pltpu.VMEM((REM,), jnp.int32),
            pltpu.VMEM((CH, 16), jnp.float32),
            pltpu.VMEM_SHARED((NP, H), jnp.float32),
            pltpu.VMEM_SHARED((NP, 16), jnp.float32),
            pltpu.SemaphoreType.DMA,
            pltpu.SemaphoreType.DMA,
        ),
        compiler_params=pltpu.CompilerParams(use_tc_tiling_on_sc=False),
    )
    return kern(zmsg, src3, typ3, dst3, srcr, typr, dstr, zrows, zdeg, ones)


def _ep_body(p0_ref, p1_ref, d_ref, zroot_ref, bias_ref, out_ref):
    inv = 1.0 / jnp.maximum(d_ref[...], 1.0)
    msg = jnp.concatenate([p0_ref[...], p1_ref[...]], axis=1)
    out_ref[...] = msg * inv + zroot_ref[...] + bias_ref[...]


def _epilogue(p0, p1, d, zroot, bias2d):
    blk = 1000
    grid = (N // blk,)
    return pl.pallas_call(
        _ep_body,
        grid=grid,
        in_specs=[
            pl.BlockSpec((blk, H), lambda i: (i, 0)),
            pl.BlockSpec((blk, H), lambda i: (i, 0)),
            pl.BlockSpec((blk, 1), lambda i: (i, 0)),
            pl.BlockSpec((blk, D), lambda i: (i, 0)),
            pl.BlockSpec((1, D), lambda i: (0, 0)),
        ],
        out_specs=pl.BlockSpec((blk, D), lambda i: (i, 0)),
        out_shape=jax.ShapeDtypeStruct((N, D), jnp.float32),
    )(p0, p1, d, zroot, bias2d)


@jax.jit
def kernel(x, edge_index, edge_type, weight, comp, bias):
    z2, zroot = _relation_transform(x, weight, comp)
    zmsg = z2.reshape(NC * N * R, H)

    src = edge_index[0].reshape(NS, EPT)
    dst = edge_index[1].reshape(NS, EPT)
    typ = edge_type.reshape(NS, EPT)
    src3 = src[:, :NCHUNK * CH].reshape(NS, NCHUNK, CH)
    dst3 = dst[:, :NCHUNK * CH].reshape(NS, NCHUNK, CH)
    typ3 = typ[:, :NCHUNK * CH].reshape(NS, NCHUNK, CH)
    srcr = src[:, NCHUNK * CH:]
    dstr = dst[:, NCHUNK * CH:]
    typr = typ[:, NCHUNK * CH:]

    zrows = jnp.zeros((RPT, H), jnp.float32)
    zdeg = jnp.zeros((RPT, 16), jnp.float32)
    ones = jnp.ones((CH, 16), jnp.float32)

    partials, degp = _sc_aggregate(zmsg, src3, typ3, dst3,
                                   srcr, typr, dstr, zrows, zdeg, ones)

    return _epilogue(partials[0, :N], partials[1, :N], degp[:N, :1],
                     zroot, bias.reshape(1, D))
